# baseline (speedup 1.0000x reference)
import jax
import jax.numpy as jnp
from jax.experimental import pallas as pl
from jax.experimental.pallas import tpu as pltpu

N = 100000
D_IN = 128
D_HID = 256
D_OUT = 128
BLOCK = 4000
NBUF = 6


def _outer(x_hbm, w1_ref, b1_ref, w2_ref, b2_ref, out_hbm):
    w1b = w1_ref[...].astype(jnp.bfloat16)
    w2b = w2_ref[...].astype(jnp.bfloat16)
    b1v = b1_ref[...]
    b2v = b2_ref[...]

    def inner(x_ref, out_ref):
        xb = x_ref[...].astype(jnp.bfloat16)
        h = jnp.dot(xb, w1b, preferred_element_type=jnp.float32)
        h = jnp.maximum(h + b1v, 0.0).astype(jnp.bfloat16)
        out = jnp.dot(h, w2b, preferred_element_type=jnp.float32)
        out_ref[...] = out + b2v

    pltpu.emit_pipeline(
        inner,
        grid=(N // BLOCK,),
        in_specs=[
            pl.BlockSpec(
                (BLOCK, D_IN), lambda i: (i, 0),
                pipeline_mode=pl.Buffered(buffer_count=NBUF),
            )
        ],
        out_specs=[pl.BlockSpec((BLOCK, D_OUT), lambda i: (i, 0))],
    )(x_hbm, out_hbm)


@jax.jit
def kernel(x, W1, b1, W2, b2):
    b1r = b1.reshape(1, D_HID)
    b2r = b2.reshape(1, D_OUT)
    return pl.pallas_call(
        _outer,
        in_specs=[
            pl.BlockSpec(memory_space=pltpu.MemorySpace.HBM),
            pl.BlockSpec(memory_space=pltpu.MemorySpace.VMEM),
            pl.BlockSpec(memory_space=pltpu.MemorySpace.VMEM),
            pl.BlockSpec(memory_space=pltpu.MemorySpace.VMEM),
            pl.BlockSpec(memory_space=pltpu.MemorySpace.VMEM),
        ],
        out_specs=pl.BlockSpec(memory_space=pltpu.MemorySpace.HBM),
        out_shape=jax.ShapeDtypeStruct((N, D_OUT), jnp.float32),
    )(x, W1, b1r, W2, b2r)


# speedup vs baseline: 1.0666x; 1.0666x over previous
import jax
import jax.numpy as jnp
from jax.experimental import pallas as pl
from jax.experimental.pallas import tpu as pltpu

N = 100000
D_IN = 128
D_HID = 256
D_OUT = 128
BLOCK = 10000
NBUF = 4


def _outer(x_hbm, w1_ref, b1_ref, w2_ref, b2_ref, out_hbm):
    w1b = w1_ref[...].astype(jnp.bfloat16)
    w2b = w2_ref[...].astype(jnp.bfloat16)
    b1v = b1_ref[...]
    b2v = b2_ref[...]

    def inner(x_ref, out_ref):
        xb = x_ref[...].astype(jnp.bfloat16)
        h = jnp.dot(xb, w1b, preferred_element_type=jnp.float32)
        h = jnp.maximum(h + b1v, 0.0).astype(jnp.bfloat16)
        out = jnp.dot(h, w2b, preferred_element_type=jnp.float32)
        out_ref[...] = out + b2v

    pltpu.emit_pipeline(
        inner,
        grid=(N // BLOCK,),
        in_specs=[
            pl.BlockSpec(
                (BLOCK, D_IN), lambda i: (i, 0),
                pipeline_mode=pl.Buffered(buffer_count=NBUF),
            )
        ],
        out_specs=[pl.BlockSpec((BLOCK, D_OUT), lambda i: (i, 0))],
    )(x_hbm, out_hbm)


@jax.jit
def kernel(x, W1, b1, W2, b2):
    b1r = b1.reshape(1, D_HID)
    b2r = b2.reshape(1, D_OUT)
    return pl.pallas_call(
        _outer,
        in_specs=[
            pl.BlockSpec(memory_space=pltpu.MemorySpace.HBM),
            pl.BlockSpec(memory_space=pltpu.MemorySpace.VMEM),
            pl.BlockSpec(memory_space=pltpu.MemorySpace.VMEM),
            pl.BlockSpec(memory_space=pltpu.MemorySpace.VMEM),
            pl.BlockSpec(memory_space=pltpu.MemorySpace.VMEM),
        ],
        out_specs=pl.BlockSpec(memory_space=pltpu.MemorySpace.HBM),
        out_shape=jax.ShapeDtypeStruct((N, D_OUT), jnp.float32),
    )(x, W1, b1r, W2, b2r)
